# Initial kernel scaffold; baseline (speedup 1.0000x reference)
import jax
import jax.numpy as jnp
from jax.experimental import pallas as pl


def kernel(x, edge_index, W_rel, b_rel, W_root, Wq, bq, Wk, bk, Wv, bv, w_sel, W_fc, b_fc, gamma, beta):
    raise NotImplementedError("write your pallas kernel here")


# speedup vs baseline: 1.0221x; 1.0221x over previous
import functools
import math

import jax
import jax.numpy as jnp
from jax import lax
from jax.experimental import pallas as pl
from jax.experimental.pallas import tpu as pltpu

N = 10000
D = 128
K_SEL = 5000
_BLK = 1000


def _dense_body(agg_ref, x_ref, wrel_ref, brel_ref, wroot_ref, wv_ref,
                bv_ref, wsel_ref, xo_ref, raw_ref):
    agg = agg_ref[...]
    xv = x_ref[...]
    h = (lax.dot_general(agg, wrel_ref[...], (((1,), (1,)), ((), ())),
                         preferred_element_type=jnp.float32)
         + brel_ref[...][None, :]
         + lax.dot_general(xv, wroot_ref[...], (((1,), (1,)), ((), ())),
                           preferred_element_type=jnp.float32))
    xo = (lax.dot_general(h, wv_ref[...], (((1,), (1,)), ((), ())),
                          preferred_element_type=jnp.float32)
          + bv_ref[...][None, :])
    xo_ref[...] = xo
    raw_ref[...] = jnp.sum(xo * wsel_ref[0, 0], axis=1, keepdims=True)


@jax.jit
def _dense_stage(agg, x, W_rel, b_rel, W_root, Wv, bv, w_sel):
    grid = N // _BLK
    xo, raw = pl.pallas_call(
        _dense_body,
        grid=(grid,),
        in_specs=[
            pl.BlockSpec((_BLK, D), lambda i: (i, 0)),
            pl.BlockSpec((_BLK, D), lambda i: (i, 0)),
            pl.BlockSpec((D, D), lambda i: (0, 0)),
            pl.BlockSpec((D,), lambda i: (0,)),
            pl.BlockSpec((D, D), lambda i: (0, 0)),
            pl.BlockSpec((D, D), lambda i: (0, 0)),
            pl.BlockSpec((D,), lambda i: (0,)),
            pl.BlockSpec((1, 1), lambda i: (0, 0), memory_space=pltpu.SMEM),
        ],
        out_specs=[
            pl.BlockSpec((_BLK, D), lambda i: (i, 0)),
            pl.BlockSpec((_BLK, 1), lambda i: (i, 0)),
        ],
        out_shape=[
            jax.ShapeDtypeStruct((N, D), jnp.float32),
            jax.ShapeDtypeStruct((N, 1), jnp.float32),
        ],
    )(agg, x, W_rel, b_rel, W_root, Wv, bv, w_sel)
    return xo, raw[:, 0]


def _fc_body(xs_ref, wfc_ref, bfc_ref, y0_ref, s_ref, s2_ref):
    i = pl.program_id(0)
    xs = xs_ref[...]
    y0 = (lax.dot_general(xs, wfc_ref[...], (((1,), (1,)), ((), ())),
                          preferred_element_type=jnp.float32)
          + bfc_ref[...][None, :])
    y0_ref[...] = y0
    s_ref[pl.ds(i, 1), :] = jnp.sum(y0, axis=0, keepdims=True)
    s2_ref[pl.ds(i, 1), :] = jnp.sum(y0 * y0, axis=0, keepdims=True)


@jax.jit
def _fc_stage(xs, W_fc, b_fc):
    grid = K_SEL // _BLK
    y0, s, s2 = pl.pallas_call(
        _fc_body,
        grid=(grid,),
        in_specs=[
            pl.BlockSpec((_BLK, D), lambda i: (i, 0)),
            pl.BlockSpec((D, D), lambda i: (0, 0)),
            pl.BlockSpec((D,), lambda i: (0,)),
        ],
        out_specs=[
            pl.BlockSpec((_BLK, D), lambda i: (i, 0)),
            pl.BlockSpec((K_SEL // _BLK, D), lambda i: (0, 0)),
            pl.BlockSpec((K_SEL // _BLK, D), lambda i: (0, 0)),
        ],
        out_shape=[
            jax.ShapeDtypeStruct((K_SEL, D), jnp.float32),
            jax.ShapeDtypeStruct((grid, D), jnp.float32),
            jax.ShapeDtypeStruct((grid, D), jnp.float32),
        ],
    )(xs, W_fc, b_fc)
    return y0, s, s2


def _bn_body(y0_ref, mu_ref, inv_ref, beta_ref, y_ref):
    y0 = y0_ref[...]
    y = (y0 - mu_ref[...]) * inv_ref[...] + beta_ref[...]
    y_ref[...] = jnp.maximum(y, 0.0)


@jax.jit
def _bn_stage(y0, mu, inv, beta):
    grid = K_SEL // _BLK
    return pl.pallas_call(
        _bn_body,
        grid=(grid,),
        in_specs=[
            pl.BlockSpec((_BLK, D), lambda i: (i, 0)),
            pl.BlockSpec((1, D), lambda i: (0, 0)),
            pl.BlockSpec((1, D), lambda i: (0, 0)),
            pl.BlockSpec((1, D), lambda i: (0, 0)),
        ],
        out_specs=pl.BlockSpec((_BLK, D), lambda i: (i, 0)),
        out_shape=jax.ShapeDtypeStruct((K_SEL, D), jnp.float32),
    )(y0, mu[None, :], inv[None, :], beta[None, :])


def kernel(x, edge_index, W_rel, b_rel, W_root, Wq, bq, Wk, bk, Wv, bv,
           w_sel, W_fc, b_fc, gamma, beta):
    n = x.shape[0]
    src, dst = edge_index[0], edge_index[1]
    agg = jax.ops.segment_sum(x[src], dst, num_segments=n)

    xo, raw = _dense_stage(agg, x, W_rel, b_rel, W_root, Wv, bv, w_sel)
    del raw

    score = jnp.tanh((xo * w_sel).sum(axis=-1) / jnp.linalg.norm(w_sel))
    k = K_SEL
    topv, perm = jax.lax.top_k(score, k)

    mapping = jnp.full((n,), -1, dtype=jnp.int32).at[perm].set(
        jnp.arange(k, dtype=jnp.int32))
    keep = jnp.sort(perm)
    new_edge = mapping[keep][None, :]
    batch_sel = jnp.zeros((k,), jnp.int32)
    attn = jnp.ones((1, n, 1, 1), jnp.float32)

    xs = xo[perm] * topv[:, None]
    y0, s, s2 = _fc_stage(xs, W_fc, b_fc)
    mu = jnp.sum(s, axis=0) / k
    var = jnp.sum(s2, axis=0) / k - mu * mu
    inv = gamma / jnp.sqrt(var + 1e-5)
    y = _bn_stage(y0, mu, inv, beta)

    return (y, new_edge, batch_sel, perm, topv, attn)
